# baseline (device time: 206375 ns/iter reference)
import jax
import jax.numpy as jnp
from jax import lax
from jax.experimental import pallas as pl
from jax.experimental.pallas import tpu as pltpu

N_DEV = 4


def kernel(O, Wo):
    B, S, H, D = O.shape
    K = H * D
    M = Wo.shape[1]
    Mh = M // 2
    S_out = S // N_DEV
    O3 = O.reshape(B, S, K)
    Wob = Wo.astype(jnp.bfloat16)

    def body(o_hbm, wo_ref, out_hbm, comm_cw, comm_ccw, ostage, pbuf_cw,
             pbuf_ccw, load_sem, store_sems, send_cw, recv_cw, send_ccw,
             recv_ccw, credit_cw, credit_ccw):
        my = lax.axis_index("i")
        left = (my + N_DEV - 1) % N_DEV
        right = (my + 1) % N_DEV

        barrier = pltpu.get_barrier_semaphore()
        for nbr in (left, right):
            pl.semaphore_signal(barrier, inc=1, device_id=(nbr,),
                                device_id_type=pl.DeviceIdType.MESH)
        pl.semaphore_wait(barrier, 2)

        def compute_partials(c_cw, c_ccw):
            for c, dst, lo in ((c_cw, pbuf_cw, 0), (c_ccw, pbuf_ccw, Mh)):
                cp = pltpu.make_async_copy(
                    o_hbm.at[:, pl.ds(c * S_out, S_out), :], ostage, load_sem)
                cp.start()
                cp.wait()
                for b in range(B):
                    dst[b, :, :] = jnp.dot(
                        ostage[b, :, :].astype(jnp.bfloat16),
                        wo_ref[:, lo:lo + Mh],
                        preferred_element_type=jnp.float32)

        compute_partials((my + N_DEV - 1) % N_DEV, (my + 1) % N_DEV)
        for b in range(B):
            comm_cw[0, b] = pbuf_cw[b, :, :].astype(jnp.bfloat16)
            comm_ccw[0, b] = pbuf_ccw[b, :, :].astype(jnp.bfloat16)

        for h in range(N_DEV - 1):
            s = h % 2
            r = (h + 1) % 2
            if h >= 1:
                pl.semaphore_wait(credit_cw, 1)
                pl.semaphore_wait(credit_ccw, 1)
            rdma_cw = pltpu.make_async_remote_copy(
                src_ref=comm_cw.at[s],
                dst_ref=comm_cw.at[r],
                send_sem=send_cw.at[h],
                recv_sem=recv_cw.at[h],
                device_id=(right,),
                device_id_type=pl.DeviceIdType.MESH,
            )
            rdma_ccw = pltpu.make_async_remote_copy(
                src_ref=comm_ccw.at[s],
                dst_ref=comm_ccw.at[r],
                send_sem=send_ccw.at[h],
                recv_sem=recv_ccw.at[h],
                device_id=(left,),
                device_id_type=pl.DeviceIdType.MESH,
            )
            rdma_cw.start()
            rdma_ccw.start()
            compute_partials((my + 2 - h) % N_DEV, (my + 2 + h) % N_DEV)
            rdma_cw.wait_send()
            rdma_ccw.wait_send()
            if h < N_DEV - 2:
                pl.semaphore_signal(credit_cw, inc=1, device_id=(left,),
                                    device_id_type=pl.DeviceIdType.MESH)
                pl.semaphore_signal(credit_ccw, inc=1, device_id=(right,),
                                    device_id_type=pl.DeviceIdType.MESH)
            stores = []
            for i, (rdma, comm, pbuf, lo) in enumerate((
                    (rdma_cw, comm_cw, pbuf_cw, 0),
                    (rdma_ccw, comm_ccw, pbuf_ccw, Mh))):
                rdma.wait_recv()
                for b in range(B):
                    acc = comm[r, b].astype(jnp.float32) + pbuf[b, :, :]
                    if h == N_DEV - 2:
                        pbuf[b, :, :] = acc
                    else:
                        comm[r, b] = acc.astype(jnp.bfloat16)
                if h == N_DEV - 2:
                    st = pltpu.make_async_copy(
                        pbuf, out_hbm.at[:, :, pl.ds(lo, Mh)],
                        store_sems.at[i])
                    st.start()
                    stores.append(st)
        for st in stores:
            st.wait()

    return pl.pallas_call(
        body,
        out_shape=jax.ShapeDtypeStruct((B, S_out, M), jnp.float32),
        in_specs=[
            pl.BlockSpec(memory_space=pl.ANY),
            pl.BlockSpec(memory_space=pltpu.VMEM),
        ],
        out_specs=pl.BlockSpec(memory_space=pl.ANY),
        scratch_shapes=[
            pltpu.VMEM((2, B, S_out, Mh), jnp.bfloat16),
            pltpu.VMEM((2, B, S_out, Mh), jnp.bfloat16),
            pltpu.VMEM((B, S_out, K), jnp.float32),
            pltpu.VMEM((B, S_out, Mh), jnp.float32),
            pltpu.VMEM((B, S_out, Mh), jnp.float32),
            pltpu.SemaphoreType.DMA,
            pltpu.SemaphoreType.DMA((2,)),
            pltpu.SemaphoreType.DMA((N_DEV - 1,)),
            pltpu.SemaphoreType.DMA((N_DEV - 1,)),
            pltpu.SemaphoreType.DMA((N_DEV - 1,)),
            pltpu.SemaphoreType.DMA((N_DEV - 1,)),
            pltpu.SemaphoreType.REGULAR,
            pltpu.SemaphoreType.REGULAR,
        ],
        compiler_params=pltpu.CompilerParams(
            collective_id=0,
            vmem_limit_bytes=64 * 1024 * 1024,
        ),
    )(O3, Wob)


# device time: 204654 ns/iter; 1.0084x vs baseline; 1.0084x over previous
import jax
import jax.numpy as jnp
from jax import lax
from jax.experimental import pallas as pl
from jax.experimental.pallas import tpu as pltpu

N_DEV = 4


def kernel(O, Wo):
    B, S, H, D = O.shape
    K = H * D
    M = Wo.shape[1]
    Mh = M // 2
    S_out = S // N_DEV
    O3 = O.reshape(B, S, K)
    Wob = Wo.astype(jnp.bfloat16)

    def body(o_hbm, wo_ref, out_hbm, comm_cw, comm_ccw, ostage, pbuf_cw,
             pbuf_ccw, load_sem, store_sems, send_cw, recv_cw, send_ccw,
             recv_ccw, credit_cw, credit_ccw):
        my = lax.axis_index("i")
        left = (my + N_DEV - 1) % N_DEV
        right = (my + 1) % N_DEV

        barrier = pltpu.get_barrier_semaphore()
        for nbr in (left, right):
            pl.semaphore_signal(barrier, inc=1, device_id=(nbr,),
                                device_id_type=pl.DeviceIdType.MESH)
        pl.semaphore_wait(barrier, 2)

        def compute_partials(c_cw, c_ccw):
            for c, dst, lo in ((c_cw, pbuf_cw, 0), (c_ccw, pbuf_ccw, Mh)):
                cp = pltpu.make_async_copy(
                    o_hbm.at[:, pl.ds(c * S_out, S_out), :], ostage, load_sem)
                cp.start()
                cp.wait()
                for b in range(B):
                    dst[b, :, :] = jnp.dot(
                        ostage[b, :, :].astype(jnp.bfloat16),
                        wo_ref[:, lo:lo + Mh],
                        preferred_element_type=jnp.float32)

        compute_partials((my + N_DEV - 1) % N_DEV, (my + 1) % N_DEV)
        for b in range(B):
            comm_cw[0, b] = pbuf_cw[b, :, :].astype(jnp.bfloat16)
            comm_ccw[0, b] = pbuf_ccw[b, :, :].astype(jnp.bfloat16)

        NSUB = 2
        S_sub = S_out // NSUB
        for h in range(N_DEV - 1):
            s = h % 2
            r = (h + 1) % 2
            subs = []
            for q in range(NSUB):
                rows = pl.ds(q * S_sub, S_sub)
                for ring, (comm, snd, rcv, credit, dst) in enumerate((
                        (comm_cw, send_cw, recv_cw, credit_cw, right),
                        (comm_ccw, send_ccw, recv_ccw, credit_ccw, left))):
                    if h >= 1:
                        pl.semaphore_wait(credit, 1)
                    rdma = pltpu.make_async_remote_copy(
                        src_ref=comm.at[s, :, rows, :],
                        dst_ref=comm.at[r, :, rows, :],
                        send_sem=snd.at[h, q],
                        recv_sem=rcv.at[h, q],
                        device_id=(dst,),
                        device_id_type=pl.DeviceIdType.MESH,
                    )
                    rdma.start()
                    subs.append(rdma)
            compute_partials((my + 2 - h) % N_DEV, (my + 2 + h) % N_DEV)
            for rdma in subs:
                rdma.wait_send()
            if h < N_DEV - 2:
                pl.semaphore_signal(credit_cw, inc=NSUB, device_id=(left,),
                                    device_id_type=pl.DeviceIdType.MESH)
                pl.semaphore_signal(credit_ccw, inc=NSUB, device_id=(right,),
                                    device_id_type=pl.DeviceIdType.MESH)
            stores = []
            for q in range(NSUB):
                rows = pl.ds(q * S_sub, S_sub)
                for ring, (comm, pbuf, lo) in enumerate((
                        (comm_cw, pbuf_cw, 0), (comm_ccw, pbuf_ccw, Mh))):
                    subs[q * 2 + ring].wait_recv()
                    for b in range(B):
                        acc = (comm[r, b, rows, :].astype(jnp.float32)
                               + pbuf[b, rows, :])
                        if h == N_DEV - 2:
                            pbuf[b, rows, :] = acc
                        else:
                            comm[r, b, rows, :] = acc.astype(jnp.bfloat16)
                    if h == N_DEV - 2:
                        st = pltpu.make_async_copy(
                            pbuf.at[:, rows, :],
                            out_hbm.at[:, rows, pl.ds(lo, Mh)],
                            store_sems.at[q, ring])
                        st.start()
                        stores.append(st)
        for st in stores:
            st.wait()

    return pl.pallas_call(
        body,
        out_shape=jax.ShapeDtypeStruct((B, S_out, M), jnp.float32),
        in_specs=[
            pl.BlockSpec(memory_space=pl.ANY),
            pl.BlockSpec(memory_space=pltpu.VMEM),
        ],
        out_specs=pl.BlockSpec(memory_space=pl.ANY),
        scratch_shapes=[
            pltpu.VMEM((2, B, S_out, Mh), jnp.bfloat16),
            pltpu.VMEM((2, B, S_out, Mh), jnp.bfloat16),
            pltpu.VMEM((B, S_out, K), jnp.float32),
            pltpu.VMEM((B, S_out, Mh), jnp.float32),
            pltpu.VMEM((B, S_out, Mh), jnp.float32),
            pltpu.SemaphoreType.DMA,
            pltpu.SemaphoreType.DMA((2, 2)),
            pltpu.SemaphoreType.DMA((N_DEV - 1, 2)),
            pltpu.SemaphoreType.DMA((N_DEV - 1, 2)),
            pltpu.SemaphoreType.DMA((N_DEV - 1, 2)),
            pltpu.SemaphoreType.DMA((N_DEV - 1, 2)),
            pltpu.SemaphoreType.REGULAR,
            pltpu.SemaphoreType.REGULAR,
        ],
        compiler_params=pltpu.CompilerParams(
            collective_id=0,
            vmem_limit_bytes=64 * 1024 * 1024,
        ),
    )(O3, Wob)
